# baseline (speedup 1.0000x reference)
import functools

import jax
import jax.numpy as jnp
from jax import lax
from jax.experimental import pallas as pl
from jax.experimental.pallas import tpu as pltpu
from jax.experimental.pallas import tpu_sc as plsc

VOCAB = 1000000
EMB_DIM = 64
BATCH = 16384

NC = 2
B_PER_C = BATCH // NC
K = 16

_mesh = plsc.ScalarSubcoreMesh(axis_name="c", num_cores=NC)


@functools.partial(
    pl.kernel,
    mesh=_mesh,
    out_type=jax.ShapeDtypeStruct((BATCH, EMB_DIM), jnp.float32),
    scratch_types=[
        pltpu.SMEM((B_PER_C,), jnp.int32),
        pltpu.SemaphoreType.DMA,
    ],
)
def _gather_rows(table_hbm, idx_hbm, out_hbm, idx_s, sem):
    base = lax.axis_index("c") * B_PER_C
    pltpu.sync_copy(idx_hbm.at[pl.ds(base, B_PER_C)], idx_s)

    def chunk(g, carry):
        row0 = g * K
        copies = [
            pltpu.async_copy(
                table_hbm.at[idx_s[row0 + j]],
                out_hbm.at[base + row0 + j],
                sem,
            )
            for j in range(K)
        ]
        for c in copies:
            c.wait()
        return carry

    lax.fori_loop(0, B_PER_C // K, chunk, 0)


def kernel(indices, table):
    return _gather_rows(table, indices.astype(jnp.int32))


# speedup vs baseline: 1.6035x; 1.6035x over previous
import functools

import jax
import jax.numpy as jnp
from jax import lax
from jax.experimental import pallas as pl
from jax.experimental.pallas import tpu as pltpu
from jax.experimental.pallas import tpu_sc as plsc

VOCAB = 1000000
EMB_DIM = 64
BATCH = 16384
LINE_W = 128

NC = 2
NS = 16
NW = NC * NS
B_PER_W = BATCH // NW
CHUNK = 128
NCHUNK = B_PER_W // CHUNK
L = 16

_mesh = plsc.VectorSubcoreMesh(core_axis_name="c", subcore_axis_name="s")


@functools.partial(
    pl.kernel,
    mesh=_mesh,
    out_type=jax.ShapeDtypeStruct((BATCH, EMB_DIM), jnp.float32),
    scratch_types=[
        pltpu.VMEM((B_PER_W,), jnp.int32),
        pltpu.VMEM((B_PER_W,), jnp.int32),
        pltpu.VMEM((2 * CHUNK, LINE_W), jnp.float32),
        pltpu.VMEM((B_PER_W, EMB_DIM), jnp.float32),
        pltpu.SemaphoreType.DMA,
    ],
    compiler_params=pltpu.CompilerParams(needs_layout_passes=False),
)
def _gather_rows(table_hbm, idx_hbm, out_hbm, idx_v, lid_v, lines_v, rows_v,
                 sem):
    wid = lax.axis_index("s") * NC + lax.axis_index("c")
    base = wid * B_PER_W
    pltpu.sync_copy(idx_hbm.at[pl.ds(base, B_PER_W)], idx_v)

    for g in range(B_PER_W // L):
        lid_v[pl.ds(g * L, L)] = lax.shift_right_logical(
            idx_v[pl.ds(g * L, L)], 1
        )

    def start(c):
        return pltpu.async_copy(
            table_hbm.at[lid_v.at[pl.ds(c * CHUNK, CHUNK)]],
            lines_v.at[pl.ds((c % 2) * CHUNK, CHUNK)],
            sem,
        )

    iota = lax.iota(jnp.int32, L)
    copies = [start(0), start(1)]
    for c in range(NCHUNK):
        copies[c].wait()
        def group(g, carry):
            row0 = c * CHUNK + g * L
            svec = iota + ((c % 2) * CHUNK + g * L)
            dvec = iota + row0
            half = lax.shift_left(
                lax.bitwise_and(idx_v[pl.ds(row0, L)], 1), 6
            )
            for c0 in range(EMB_DIM):
                cvec = lax.bitwise_and(iota + c0, EMB_DIM - 1)
                vals = plsc.load_gather(lines_v, [svec, half + cvec])
                plsc.store_scatter(rows_v, [dvec, cvec], vals)
            return carry
        lax.fori_loop(0, CHUNK // L, group, 0)
        if c + 2 < NCHUNK:
            copies.append(start(c + 2))

    pltpu.sync_copy(rows_v, out_hbm.at[pl.ds(base, B_PER_W)])


def kernel(indices, table):
    table2 = table.reshape(VOCAB // 2, LINE_W)
    return _gather_rows(table2, indices.astype(jnp.int32))


# speedup vs baseline: 1.6503x; 1.0292x over previous
import functools

import jax
import jax.numpy as jnp
from jax import lax
from jax.experimental import pallas as pl
from jax.experimental.pallas import tpu as pltpu
from jax.experimental.pallas import tpu_sc as plsc

VOCAB = 1000000
EMB_DIM = 64
BATCH = 16384

NC = 2
NS = 16
NW = NC * NS
B_PER_W = BATCH // NW
K = 16

_mesh = plsc.VectorSubcoreMesh(core_axis_name="c", subcore_axis_name="s")


@functools.partial(
    pl.kernel,
    mesh=_mesh,
    out_type=jax.ShapeDtypeStruct((BATCH, EMB_DIM), jnp.float32),
    scratch_types=[
        pltpu.VMEM_SHARED((BATCH,), jnp.int32),
        pltpu.SMEM((B_PER_W,), jnp.int32),
        pltpu.SemaphoreType.DMA,
    ],
)
def _gather_rows(table_hbm, idx_hbm, out_hbm, idx_sp, idx_s, sem):
    cid = lax.axis_index("c")
    sid = lax.axis_index("s")
    wid = sid * NC + cid
    base = wid * B_PER_W

    @pl.when(sid == 0)
    def _():
        pltpu.sync_copy(idx_hbm, idx_sp)

    plsc.subcore_barrier()
    pltpu.sync_copy(idx_sp.at[pl.ds(base, B_PER_W)], idx_s)

    def chunk(g, carry):
        row0 = g * K
        copies = [
            pltpu.async_copy(
                table_hbm.at[idx_s[row0 + j]],
                out_hbm.at[base + row0 + j],
                sem,
            )
            for j in range(K)
        ]
        for cp in copies:
            cp.wait()
        return carry

    lax.fori_loop(0, B_PER_W // K, chunk, 0)


def kernel(indices, table):
    return _gather_rows(table, indices.astype(jnp.int32))


# speedup vs baseline: 2.6350x; 1.5966x over previous
import functools

import jax
import jax.numpy as jnp
from jax import lax
from jax.experimental import pallas as pl
from jax.experimental.pallas import tpu as pltpu
from jax.experimental.pallas import tpu_sc as plsc

VOCAB = 1000000
EMB_DIM = 64
BATCH = 16384

NC = 2
NS = 16
NW = NC * NS
B_PER_W = BATCH // NW
K = 16

_mesh = plsc.VectorSubcoreMesh(core_axis_name="c", subcore_axis_name="s")


@functools.partial(
    pl.kernel,
    mesh=_mesh,
    out_type=jax.ShapeDtypeStruct((BATCH, EMB_DIM), jnp.float32),
    scratch_types=[
        pltpu.VMEM_SHARED((BATCH,), jnp.int32),
        pltpu.SMEM((B_PER_W,), jnp.int32),
        pltpu.VMEM((B_PER_W, EMB_DIM), jnp.float32),
        pltpu.SemaphoreType.DMA,
    ],
)
def _gather_rows(table_hbm, idx_hbm, out_hbm, idx_sp, idx_s, rows_v, sem):
    cid = lax.axis_index("c")
    sid = lax.axis_index("s")
    wid = sid * NC + cid
    base = wid * B_PER_W

    @pl.when(sid == 0)
    def _():
        pltpu.sync_copy(idx_hbm, idx_sp)

    plsc.subcore_barrier()
    pltpu.sync_copy(idx_sp.at[pl.ds(base, B_PER_W)], idx_s)

    def chunk(g, carry):
        row0 = g * K
        copies = [
            pltpu.async_copy(
                table_hbm.at[idx_s[row0 + j]],
                rows_v.at[row0 + j],
                sem,
            )
            for j in range(K)
        ]
        for cp in copies:
            cp.wait()
        return carry

    lax.fori_loop(0, B_PER_W // K, chunk, 0)
    pltpu.sync_copy(rows_v, out_hbm.at[pl.ds(base, B_PER_W)])


def kernel(indices, table):
    return _gather_rows(table, indices.astype(jnp.int32))
